# baseline (speedup 1.0000x reference)
import functools

import jax
import jax.numpy as jnp
from jax import lax
from jax.experimental import pallas as pl
from jax.experimental.pallas import tpu as pltpu
from jax.experimental.pallas import tpu_sc as plsc

N = 100000
D = 64
B = 16384
L = 16
NC = 2
NS = 16
NW = NC * NS
CPT = N // NW
NQ = 4
DQ = D // NQ
CH = 128
NGA = B // L
NGC = 196
CPAD = NGC * L
SROWS = CPT + 1
MCAP = B + CH
PCAP = CPT + CH
PCAPR = ((PCAP + L - 1) // L) * L
COPYR = 625
DUMMY = CPT * 16384


def _bcast_lane(v, j):
    idx = jnp.full((L, 1), j, dtype=jnp.int32)
    dn = lax.GatherDimensionNumbers(
        offset_dims=(), collapsed_slice_dims=(0,), start_index_map=(0,))
    return lax.gather(v, idx, dn, slice_sizes=(1,),
                      mode=lax.GatherScatterMode.PROMISE_IN_BOUNDS)


def _body(y_hbm, xq_hbm, cq_hbm, ctr_hbm, out_hbm,
          y_buf, memb, counts, sums, plist, xb, cb, ib, ci, cpy, ctr_buf):
    iota = lax.iota(jnp.int32, L)
    fiota = iota.astype(jnp.float32)
    lane0 = iota == 0
    zeros = jnp.zeros((L,), jnp.float32)

    wid = lax.axis_index("s") * NC + lax.axis_index("c")
    lo = wid * CPT
    hi = lo + CPT

    ctr_buf[...] = zeros
    pltpu.sync_copy(ctr_hbm, ctr_buf.at[pl.ds(0, 1)])
    ctr_b = _bcast_lane(ctr_buf[...], 0)
    r1v = 1.0 / (ctr_b + 1.0)
    a_old_v = ctr_b * r1v

    pltpu.sync_copy(y_hbm, y_buf)

    dummy_vec = jnp.full((L,), DUMMY, jnp.int32)

    def init_memb(g, _):
        memb[pl.ds(g * L, L)] = dummy_vec
        return 0
    lax.fori_loop(0, MCAP // L, init_memb, 0)

    zvec_i = jnp.zeros((L,), jnp.int32)

    def init_plist(g, _):
        plist[pl.ds(g * L, L)] = zvec_i
        return 0
    lax.fori_loop(0, PCAPR // L, init_plist, 0)

    def init_counts(g, _):
        counts[pl.ds(g * L, L)] = zeros
        return 0
    lax.fori_loop(0, CPAD // L, init_counts, 0)

    qbase = 4 * lo

    def copy_body(k, _):
        start = qbase + k * COPYR
        pltpu.sync_copy(cq_hbm.at[pl.ds(start, COPYR)], cpy)
        pltpu.sync_copy(cpy, out_hbm.at[pl.ds(start, COPYR)])
        return 0
    lax.fori_loop(0, (4 * CPT) // COPYR, copy_body, 0)

    def phase_a(g, off):
        yv = y_buf[pl.ds(g * L, L)]
        inm = (yv >= lo) & (yv < hi)
        cl = yv - lo
        packed = cl * 16384 + (g * L + iota)
        pos = plsc.cumsum(inm.astype(jnp.int32))
        addr = off + pos - 1
        plsc.store_scatter(memb, [addr], packed, mask=inm)
        return off + jnp.max(pos)
    m_cnt = lax.fori_loop(0, NGA, phase_a, jnp.int32(0))
    nch = (m_cnt + CH - 1) // CH

    ones_f = jnp.ones((L,), jnp.float32)

    def quarter(q, p_cnt):
        def zsums(g, _):
            sums[pl.ds(g * L, L)] = zeros
            return 0
        lax.fori_loop(0, (SROWS * L) // L, zsums, 0)

        cnt_add = jnp.where(q == 0, 1.0, 0.0)
        cnt_vec = jnp.where(lane0, cnt_add, 0.0)

        def acc_chunk(k, _):
            for g8 in range(CH // L):
                sl = memb[pl.ds(k * CH + g8 * L, L)]
                bx = sl & 16383
                ib[pl.ds(g8 * L, L)] = bx * 4 + q
            pltpu.sync_copy(xq_hbm.at[ib], xb)
            for g8 in range(CH // L):
                sl = memb[pl.ds(k * CH + g8 * L, L)]
                cl = sl >> 14
                for j in range(L):
                    cjb = _bcast_lane(cl, j)
                    addr = cjb * L + iota
                    xv = xb[g8 * L + j]
                    plsc.addupdate_scatter(sums, [addr], xv)
                    plsc.addupdate_scatter(counts, [cjb], cnt_vec,
                                           mask=lane0)
            return 0
        lax.fori_loop(0, nch, acc_chunk, 0)

        def build_plist(_):
            def pgrp(g, off):
                cnts = counts[pl.ds(g * L, L)]
                cid = g * L + iota
                pres = (cnts > 0.0) & (cid < CPT)
                pos = plsc.cumsum(pres.astype(jnp.int32))
                addr = off + pos - 1
                plsc.store_scatter(plist, [addr], cid, mask=pres)
                return off + jnp.max(pos)
            return lax.fori_loop(0, NGC, pgrp, jnp.int32(0))
        p_cnt = lax.cond(q == 0, build_plist, lambda _: p_cnt, 0)
        nchp = (p_cnt + CH - 1) // CH

        def upd_chunk(t, _):
            for g8 in range(CH // L):
                pv = plist[pl.ds(t * CH + g8 * L, L)]
                ci[pl.ds(g8 * L, L)] = (pv + lo) * 4 + q
            pltpu.sync_copy(cq_hbm.at[ci], cb)
            for g8 in range(CH // L):
                pv = plist[pl.ds(t * CH + g8 * L, L)]
                cnts = plsc.load_gather(counts, [pv])
                pres = cnts > 0.0
                ssum = jnp.where(pres, r1v / jnp.maximum(cnts, 1.0), 0.0)
                sold = jnp.where(pres, a_old_v, 1.0)
                for j in range(L):
                    ssj = _bcast_lane(ssum, j)
                    soj = _bcast_lane(sold, j)
                    cjb = _bcast_lane(pv, j)
                    saddr = cjb * L + iota
                    sv = plsc.load_gather(sums, [saddr])
                    cv = cb[g8 * L + j]
                    cb[g8 * L + j] = cv * soj + sv * ssj
            pltpu.sync_copy(cb, out_hbm.at[ci])
            return 0
        lax.fori_loop(0, nchp, upd_chunk, 0)
        return p_cnt

    lax.fori_loop(0, NQ, quarter, jnp.int32(0))


@jax.jit
def _run(x, y, centers, counter):
    y32 = y.astype(jnp.int32)
    xq = x.reshape(B * 4, DQ)
    cq = centers.reshape(N * 4, DQ)
    mesh = plsc.VectorSubcoreMesh(core_axis_name="c", subcore_axis_name="s",
                                  num_cores=NC, num_subcores=NS)
    out = pl.kernel(
        _body,
        out_type=jax.ShapeDtypeStruct((N * 4, DQ), jnp.float32),
        mesh=mesh,
        compiler_params=pltpu.CompilerParams(use_tc_tiling_on_sc=False,
                                             needs_layout_passes=False),
        scratch_types=[
            pltpu.VMEM((B,), jnp.int32),
            pltpu.VMEM((MCAP,), jnp.int32),
            pltpu.VMEM((CPAD,), jnp.float32),
            pltpu.VMEM((SROWS * L,), jnp.float32),
            pltpu.VMEM((PCAPR,), jnp.int32),
            pltpu.VMEM((CH, DQ), jnp.float32),
            pltpu.VMEM((CH, DQ), jnp.float32),
            pltpu.VMEM((CH,), jnp.int32),
            pltpu.VMEM((CH,), jnp.int32),
            pltpu.VMEM((COPYR, DQ), jnp.float32),
            pltpu.VMEM((L,), jnp.float32),
        ],
    )(y32, xq, cq, counter)
    return out.reshape(N, D)


def kernel(x, y, centers, counter):
    new_centers = _run(x, y, centers, counter)
    return new_centers, counter + 1.0
